# Initial kernel scaffold; baseline (speedup 1.0000x reference)
import jax
import jax.numpy as jnp
from jax.experimental import pallas as pl


def kernel(x, params):
    raise NotImplementedError("write your pallas kernel here")


# speedup vs baseline: 6.3545x; 6.3545x over previous
import functools

import jax
import jax.numpy as jnp
from jax import lax
from jax.experimental import pallas as pl
from jax.experimental.pallas import tpu as pltpu
from jax.experimental.pallas import tpu_sc as plsc

B = 4
N = 2048
BN = B * N
KNN = 20
BLK = 256
NB = N // BLK
CP = 128
EPS = 1e-5

NEG = -3.0e38


def _prep_body(C, cast16, x_ref, xp_ref, idx_ref):
    b = pl.program_id(0)
    nb = pl.program_id(1)
    xb = x_ref[0]
    rows = x_ref[0, pl.ds(nb * BLK, BLK), :]
    sq = jnp.sum(xb * xb, axis=1, keepdims=True)
    rT = 2.0 * lax.dot_general(xb, rows, (((1,), (1,)), ((), ())),
                               preferred_element_type=jnp.float32)
    rT = rT - sq
    iota = lax.broadcasted_iota(jnp.int32, (N, BLK), 0)
    base = b * N
    for t in range(KNN):
        m = jnp.max(rT, axis=0, keepdims=True)
        sel = jnp.min(jnp.where(rT == m, iota, N), axis=0, keepdims=True)
        idx_ref[0, t:t + 1, :] = sel + base
        rT = jnp.where(iota == sel, NEG, rT)
    if C == CP:
        xp_ref[...] = rows
    else:
        xp_ref[...] = jnp.concatenate(
            [rows, jnp.zeros((BLK, CP - C), jnp.float32)], axis=1)


def _prep_call(x3d, C, cast16):
    return pl.pallas_call(
        functools.partial(_prep_body, C, cast16),
        grid=(B, NB),
        in_specs=[
            pl.BlockSpec((1, N, C), lambda b, nb: (b, 0, 0)),
        ],
        out_specs=[
            pl.BlockSpec((BLK, CP), lambda b, nb: (b * NB + nb, 0)),
            pl.BlockSpec((1, KNN, BLK), lambda b, nb: (b * NB + nb, 0, 0)),
        ],
        out_shape=[
            jax.ShapeDtypeStruct((BN, CP), jnp.float32),
            jax.ShapeDtypeStruct((BN // BLK, KNN, BLK), jnp.int32),
        ],
    )(x3d)


def _make_sc_gather():
    NW = 32
    PPW = BN // NW
    CH = 32
    NCHUNK = PPW // CH
    mesh = plsc.VectorSubcoreMesh(core_axis_name="c", subcore_axis_name="s")

    @functools.partial(
        pl.kernel, mesh=mesh,
        out_type=jax.ShapeDtypeStruct((KNN, BN, CP), jnp.float32),
        scratch_types=[
            pltpu.VMEM((KNN, PPW), jnp.int32),
            pltpu.VMEM((KNN, CH, CP), jnp.float32),
            pltpu.SemaphoreType.DMA,
        ],
    )
    def g(xp_hbm, idx_hbm, xj_hbm, idx_v, rows_v, sem):
        wid = lax.axis_index("s") * 2 + lax.axis_index("c")
        pltpu.sync_copy(idx_hbm.at[wid], idx_v)

        def chunk_body(c, carry):
            base = wid * PPW + c * CH
            cps = [pltpu.async_copy(
                       xp_hbm.at[idx_v.at[k, pl.ds(c * CH, CH)]],
                       rows_v.at[k], sem)
                   for k in range(KNN)]
            for cp in cps:
                cp.wait()
            for k in range(KNN):
                pltpu.sync_copy(rows_v.at[k], xj_hbm.at[k, pl.ds(base, CH)])
            return carry
        lax.fori_loop(0, NCHUNK, chunk_body, 0)

    return g


def _edge_body(C, O, cast16, xj_ref, xp_ref, w_ref, mx_ref, st_ref, acc_ref):
    ph = pl.program_id(0)
    i = pl.program_id(1)
    xj = xj_ref[:, :, 0:C]
    xi = xp_ref[:, 0:C]
    d = xj - xi[None, :, :]
    xib = jnp.broadcast_to(xi[None, :, :], (KNN, BLK, C))
    feat = jnp.concatenate([d, xib], axis=2)
    h = lax.dot_general(feat.reshape(KNN * BLK, 2 * C), w_ref[...],
                        (((1,), (0,)), ((), ())),
                        preferred_element_type=jnp.float32
                        ).reshape(KNN, BLK, O)

    def _ktree(a):
        s10 = a[0:10] + a[10:20]
        s5 = s10[0:5] + s10[5:10]
        s2 = s5[0:2] + s5[2:4]
        s1 = s2[0:1] + s2[1:2] + s5[4:5]
        return jnp.sum(s1[0], axis=0, keepdims=True)

    def _fold(p):
        while p.shape[0] > 1:
            half = p.shape[0] // 2
            p = p[0:half] + p[half:2 * half]
        return p

    NSTEP = BN // BLK

    @pl.when(jnp.logical_and(ph == 0, i == 0))
    def _init():
        acc_ref[...] = jnp.zeros_like(acc_ref)

    @pl.when(ph == 0)
    def _p0():
        acc_ref[pl.ds(i, 1), :] = _ktree(h)

    @pl.when(ph == 1)
    def _p1():
        ms = _fold(acc_ref[0:NSTEP, :])
        m = ms / float(BN * KNN)
        dm = h - m[None, :, :]
        acc_ref[pl.ds(NSTEP + i, 1), :] = _ktree(dm * dm)
        mx_ref[...] = jnp.max(h, axis=0)
        st_ref[0:1, :] = ms
        st_ref[1:2, :] = _fold(acc_ref[NSTEP:2 * NSTEP, :])
        st_ref[2:8, :] = jnp.zeros((6, st_ref.shape[1]), jnp.float32)


def _edge_call(XJ, XP, WF, C, O, cast16):
    return pl.pallas_call(
        functools.partial(_edge_body, C, O, cast16),
        grid=(2, BN // BLK),
        in_specs=[
            pl.BlockSpec((KNN, BLK, CP), lambda ph, i: (0, i, 0)),
            pl.BlockSpec((BLK, CP), lambda ph, i: (i, 0)),
            pl.BlockSpec((2 * C, O), lambda ph, i: (0, 0)),
        ],
        out_specs=[
            pl.BlockSpec((BLK, O), lambda ph, i: (i, 0)),
            pl.BlockSpec((8, O), lambda ph, i: (0, 0)),
        ],
        out_shape=[
            jax.ShapeDtypeStruct((BN, O), jnp.float32),
            jax.ShapeDtypeStruct((8, O), jnp.float32),
        ],
        scratch_shapes=[pltpu.VMEM((2 * (BN // BLK), O), jnp.float32)],
    )(XJ, XP, WF)


def _finish_body(O, mx_ref, st_ref, x_ref):
    cnt = float(BN * KNN)
    m = st_ref[0:1, :] / cnt
    var = st_ref[1:2, :] / cnt
    h = (mx_ref[...] - m) / jnp.sqrt(var + EPS)
    x_ref[...] = jnp.where(h >= 0, h, 0.2 * h)


def _finish_call(MX, ST, O):
    return pl.pallas_call(
        functools.partial(_finish_body, O),
        grid=(B,),
        in_specs=[pl.BlockSpec((N, O), lambda b: (b, 0)),
                  pl.BlockSpec((8, O), lambda b: (0, 0))],
        out_specs=pl.BlockSpec((N, O), lambda b: (b, 0)),
        out_shape=jax.ShapeDtypeStruct((BN, O), jnp.float32),
    )(MX, ST)


def _head_body(n_raw, norm_first, emit_stats, O, *refs):
    i = 0
    y_ref = st_ref = None
    if norm_first:
        y_ref, st_ref = refs[0], refs[1]
        i = 2
    xs = refs[i:i + n_raw]
    i += n_raw
    n_w = n_raw + (1 if norm_first else 0)
    Ws = refs[i:i + n_w]
    i += n_w
    y_out = refs[i]
    i += 1
    st_out = refs[i] if emit_stats else None
    acc = refs[i + 1] if emit_stats else None
    b = pl.program_id(0)

    terms = []
    wi = 0
    if norm_first:
        bnf = float(BN)
        m = st_ref[0:1, :] / bnf
        var = st_ref[1:2, :] / bnf - m * m
        xn = (y_ref[...] - m) / jnp.sqrt(var + EPS)
        xn = jnp.where(xn >= 0, xn, 0.2 * xn)
        terms.append(jnp.dot(xn, Ws[0][...], preferred_element_type=jnp.float32))
        wi = 1
    for j in range(n_raw):
        terms.append(jnp.dot(xs[j][...], Ws[wi + j][...],
                             preferred_element_type=jnp.float32))
    y = functools.reduce(lambda a, c: a + c, terms)
    y_out[...] = y
    if emit_stats:
        @pl.when(b == 0)
        def _init():
            acc[...] = jnp.zeros_like(acc)
        acc[0:1, :] += jnp.sum(y, axis=0, keepdims=True)
        acc[1:2, :] += jnp.sum(y * y, axis=0, keepdims=True)
        st_out[...] = acc[...]


def _head_call(y_prev, st_prev, xs, Ws, O, emit_stats):
    norm_first = y_prev is not None
    ins = []
    in_specs = []
    if norm_first:
        Cprev = y_prev.shape[1]
        ins += [y_prev, st_prev]
        in_specs += [pl.BlockSpec((N, Cprev), lambda b: (b, 0)),
                     pl.BlockSpec((8, Cprev), lambda b: (0, 0))]
    for xarr in xs:
        Ci = xarr.shape[1]
        ins.append(xarr)
        in_specs.append(pl.BlockSpec((N, Ci), lambda b: (b, 0)))
    for Warr in Ws:
        Ci = Warr.shape[0]
        ins.append(Warr)
        in_specs.append(pl.BlockSpec((Ci, O), lambda b: (0, 0)))
    out_specs = [pl.BlockSpec((N, O), lambda b: (b, 0))]
    out_shape = [jax.ShapeDtypeStruct((BN, O), jnp.float32)]
    scratch = []
    if emit_stats:
        out_specs.append(pl.BlockSpec((8, O), lambda b: (0, 0)))
        out_shape.append(jax.ShapeDtypeStruct((8, O), jnp.float32))
        scratch.append(pltpu.VMEM((8, O), jnp.float32))
    res = pl.pallas_call(
        functools.partial(_head_body, len(xs), norm_first, emit_stats, O),
        grid=(B,),
        in_specs=in_specs,
        out_specs=out_specs,
        out_shape=out_shape,
        scratch_shapes=scratch,
    )(*ins)
    return res if emit_stats else res[0]


def _stage(xf, W, C, O, cast16):
    WF = jnp.transpose(W)
    XP, IDXT = _prep_call(xf.reshape(B, N, C), C, cast16)
    XJ = _make_sc_gather()(XP, IDXT)
    MX, ST = _edge_call(XJ, XP, WF, C, O, cast16)
    return _finish_call(MX, ST, O)


def kernel(x, params):
    p = params
    xf = jnp.transpose(x, (0, 2, 1)).reshape(BN, 6)
    x1 = _stage(xf, p['W1'], 6, 64, False)
    x2 = _stage(x1, p['W2'], 64, 64, True)
    x3 = _stage(x2, p['W3'], 64, 128, True)
    x4 = _stage(x3, p['W4'], 128, 256, True)

    W5T = jnp.transpose(p['W5'])
    W5s = [W5T[0:64], W5T[64:128], W5T[128:256], W5T[256:512]]
    y5, st5 = _head_call(None, None, [x1, x2, x3, x4], W5s, 1024, True)

    W6T = jnp.transpose(p['W6'])
    W6s = [W6T[0:1024], W6T[1024:1088], W6T[1088:1152],
           W6T[1152:1280], W6T[1280:1536]]
    y6, st6 = _head_call(y5, st5, [x1, x2, x3, x4], W6s, 256, True)

    W7T = jnp.transpose(p['W7'])
    y7, st7 = _head_call(y6, st6, [], [W7T], 128, True)

    W8T = jnp.transpose(p['W8'])
    y8 = _head_call(y7, st7, [], [W8T], 2, False)

    return jnp.transpose(y8.reshape(B, N, 2), (0, 2, 1))
